# baseline (speedup 1.0000x reference)
import jax
import jax.numpy as jnp
from jax.experimental import pallas as pl
from jax.experimental.pallas import tpu as pltpu

N = 5000
D = 4096
C = 81
CW = C + 4
STR = 200
SUP = 1000
SPS = SUP // STR
NSC = N // SUP


def _head_kernel(f_hbm, p_ref, w_ref, b_ref, cls_ref, box_ref, buf, sems):
    w = w_ref[...].astype(jnp.bfloat16)
    bvec = b_ref[...]

    def start_super(s, half):
        for j in range(SPS):
            pltpu.make_async_copy(
                f_hbm.at[pl.ds(s * SUP + j * STR, STR), :],
                buf.at[half, pl.ds(j * STR, STR), :],
                sems.at[half, j]).start()

    start_super(0, 0)
    start_super(1, 1)
    for s in range(NSC):
        half = s % 2
        for j in range(SPS):
            pltpu.make_async_copy(
                f_hbm.at[pl.ds(s * SUP + j * STR, STR), :],
                buf.at[half, pl.ds(j * STR, STR), :],
                sems.at[half, j]).wait()
        acc = jnp.dot(buf[half].astype(jnp.bfloat16), w,
                      preferred_element_type=jnp.float32)
        acc = acc + bvec
        cls_ref[pl.ds(s * SUP, SUP), :] = acc[:, :C]

        d = acc[:, C:CW]
        p = p_ref[pl.ds(s * SUP, SUP), :]
        px, py, pw, ph = p[:, 0:1], p[:, 1:2], p[:, 2:3], p[:, 3:4]
        x = d[:, 0:1] * pw + px
        y = d[:, 1:2] * ph + py
        ew = jnp.exp(d[:, 2:3])
        box_ref[pl.ds(s * SUP, SUP), :] = jnp.concatenate(
            [x, y, ew * pw, ew * ph], axis=1)

        if s + 2 < NSC:
            start_super(s + 2, half)


def kernel(feats, proposals_xywh, W_cls, b_cls, W_reg, b_reg):
    w_t = jnp.concatenate([W_cls, W_reg], axis=0).T
    b = jnp.concatenate([b_cls, b_reg]).reshape(1, CW)
    cls_out, box_out = pl.pallas_call(
        _head_kernel,
        in_specs=[
            pl.BlockSpec(memory_space=pltpu.MemorySpace.HBM),
            pl.BlockSpec(memory_space=pltpu.MemorySpace.VMEM),
            pl.BlockSpec(memory_space=pltpu.MemorySpace.VMEM),
            pl.BlockSpec(memory_space=pltpu.MemorySpace.VMEM),
        ],
        out_specs=[
            pl.BlockSpec(memory_space=pltpu.MemorySpace.VMEM),
            pl.BlockSpec(memory_space=pltpu.MemorySpace.VMEM),
        ],
        out_shape=[
            jax.ShapeDtypeStruct((N, C), jnp.float32),
            jax.ShapeDtypeStruct((N, 4), jnp.float32),
        ],
        scratch_shapes=[
            pltpu.VMEM((2, SUP, D), jnp.float32),
            pltpu.SemaphoreType.DMA((2, SPS)),
        ],
    )(feats, proposals_xywh, w_t, b)
    return (cls_out, box_out)


# speedup vs baseline: 1.0896x; 1.0896x over previous
import jax
import jax.numpy as jnp
from jax.experimental import pallas as pl
from jax.experimental.pallas import tpu as pltpu

N = 5000
D = 4096
C = 81
CW = C + 4
TILE = 1000


def _head_kernel(f_ref, p_ref, w_ref, b_ref, cls_ref, box_ref):
    acc = jnp.dot(f_ref[...].astype(jnp.bfloat16),
                  w_ref[...].astype(jnp.bfloat16),
                  preferred_element_type=jnp.float32)
    acc = acc + b_ref[...]
    cls_ref[...] = acc[:, :C]

    d = acc[:, C:CW]
    p = p_ref[...]
    px, py, pw, ph = p[:, 0:1], p[:, 1:2], p[:, 2:3], p[:, 3:4]
    x = d[:, 0:1] * pw + px
    y = d[:, 1:2] * ph + py
    ew = jnp.exp(d[:, 2:3])
    box_ref[...] = jnp.concatenate([x, y, ew * pw, ew * ph], axis=1)


def kernel(feats, proposals_xywh, W_cls, b_cls, W_reg, b_reg):
    w_t = jnp.concatenate([W_cls, W_reg], axis=0).T
    b = jnp.concatenate([b_cls, b_reg]).reshape(1, CW)
    grid = (N // TILE,)
    cls_out, box_out = pl.pallas_call(
        _head_kernel,
        grid=grid,
        in_specs=[
            pl.BlockSpec((TILE, D), lambda i: (i, 0)),
            pl.BlockSpec((TILE, 4), lambda i: (i, 0)),
            pl.BlockSpec((D, CW), lambda i: (0, 0)),
            pl.BlockSpec((1, CW), lambda i: (0, 0)),
        ],
        out_specs=[
            pl.BlockSpec((TILE, C), lambda i: (i, 0)),
            pl.BlockSpec((TILE, 4), lambda i: (i, 0)),
        ],
        out_shape=[
            jax.ShapeDtypeStruct((N, C), jnp.float32),
            jax.ShapeDtypeStruct((N, 4), jnp.float32),
        ],
        compiler_params=pltpu.CompilerParams(
            dimension_semantics=("parallel",)),
    )(feats, proposals_xywh, w_t, b)
    return (cls_out, box_out)
